# Initial kernel scaffold; baseline (speedup 1.0000x reference)
import jax
import jax.numpy as jnp
from jax.experimental import pallas as pl


def kernel(embed, adj_sp_norm, edge_index, edge_weight, trend):
    raise NotImplementedError("write your pallas kernel here")


# speedup vs baseline: 1.0036x; 1.0036x over previous
import functools

import jax
import jax.numpy as jnp
from jax.experimental import pallas as pl
from jax.experimental.pallas import tpu as pltpu

_N_HOPS = 3


def _hops_kernel(x0_ref, m_ref, out_ref, state_ref):
    h = pl.program_id(0)

    @pl.when(h == 0)
    def _init():
        x0 = x0_ref[...]
        state_ref[...] = x0
        out_ref[:, 0, :] = x0

    @pl.when(h > 0)
    def _hop():
        new = jnp.dot(
            m_ref[...],
            state_ref[...].astype(jnp.bfloat16),
            preferred_element_type=jnp.float32,
        )
        out_ref[:, h, :] = new
        state_ref[...] = new


def kernel(embed, adj_sp_norm, edge_index, edge_weight, trend):
    del adj_sp_norm, edge_weight
    N, D = embed.shape
    n_pad = (N + 127) // 128 * 128
    row, col = edge_index[0], edge_index[1]

    m = (
        jnp.zeros((n_pad, n_pad), dtype=jnp.float32)
        .at[col, row].add(trend)
        .astype(jnp.bfloat16)
    )
    x0 = embed.astype(jnp.float32)
    if n_pad != N:
        x0 = jnp.pad(x0, ((0, n_pad - N), (0, 0)))

    out = pl.pallas_call(
        _hops_kernel,
        out_shape=jax.ShapeDtypeStruct((n_pad, _N_HOPS + 1, D), jnp.float32),
        grid_spec=pltpu.PrefetchScalarGridSpec(
            num_scalar_prefetch=0,
            grid=(_N_HOPS + 1,),
            in_specs=[
                pl.BlockSpec((n_pad, D), lambda h: (0, 0)),
                pl.BlockSpec((n_pad, n_pad), lambda h: (0, 0)),
            ],
            out_specs=pl.BlockSpec(
                (n_pad, _N_HOPS + 1, D), lambda h: (0, 0, 0)
            ),
            scratch_shapes=[pltpu.VMEM((n_pad, D), jnp.float32)],
        ),
        compiler_params=pltpu.CompilerParams(
            dimension_semantics=("arbitrary",),
            vmem_limit_bytes=int((64 << 20) * 0.9),
        ),
        cost_estimate=pl.CostEstimate(
            flops=2 * _N_HOPS * D * n_pad * n_pad,
            transcendentals=0,
            bytes_accessed=2 * n_pad * n_pad * 2
            + n_pad * D * 4
            + (_N_HOPS + 1) * n_pad * D * 4,
        ),
    )(x0, m)
    return out[:N].astype(embed.dtype)


# speedup vs baseline: 1.6429x; 1.6370x over previous
import functools

import jax
import jax.numpy as jnp
from jax.experimental import pallas as pl
from jax.experimental.pallas import tpu as pltpu

_N_HOPS = 3


def _hops_kernel(x0_ref, m_ref, out_ref, state_ref):
    h = pl.program_id(0)

    @pl.when(h == 0)
    def _init():
        x0 = x0_ref[...]
        state_ref[...] = x0
        out_ref[:, 0, :] = x0

    @pl.when(h > 0)
    def _hop():
        new = jnp.dot(
            m_ref[...],
            state_ref[...].astype(jnp.bfloat16),
            preferred_element_type=jnp.float32,
        )
        out_ref[:, h, :] = new
        state_ref[...] = new


def _build_message_matrix(row, col, trend, n_pad):
    lin = col * n_pad + row
    E = lin.shape[0]
    if E % 16 == 0 and E >= 16:
        q = E // 16
        sizes = [q, 2 * q, 4 * q, 4 * q, 4 * q, q]
    else:
        sizes = [E]
    m = jnp.zeros((n_pad * n_pad,), dtype=jnp.float32)
    hist = []
    off = 0
    for i, c in enumerate(sizes):
        idx = lin[off : off + c]
        val = trend[off : off + c]
        if i >= 2:
            idx, _ = jax.lax.optimization_barrier((idx, hist[i - 2]))
        m = m.at[idx].add(val)
        hist.append(m)
        off += c
    return m.reshape(n_pad, n_pad)


def kernel(embed, adj_sp_norm, edge_index, edge_weight, trend):
    del adj_sp_norm, edge_weight
    N, D = embed.shape
    n_pad = (N + 127) // 128 * 128
    row, col = edge_index[0], edge_index[1]

    m = _build_message_matrix(row, col, trend, n_pad).astype(jnp.bfloat16)
    x0 = embed.astype(jnp.float32)
    if n_pad != N:
        x0 = jnp.pad(x0, ((0, n_pad - N), (0, 0)))

    out = pl.pallas_call(
        _hops_kernel,
        out_shape=jax.ShapeDtypeStruct((n_pad, _N_HOPS + 1, D), jnp.float32),
        grid_spec=pltpu.PrefetchScalarGridSpec(
            num_scalar_prefetch=0,
            grid=(_N_HOPS + 1,),
            in_specs=[
                pl.BlockSpec((n_pad, D), lambda h: (0, 0)),
                pl.BlockSpec((n_pad, n_pad), lambda h: (0, 0)),
            ],
            out_specs=pl.BlockSpec(
                (n_pad, _N_HOPS + 1, D), lambda h: (0, 0, 0)
            ),
            scratch_shapes=[pltpu.VMEM((n_pad, D), jnp.float32)],
        ),
        compiler_params=pltpu.CompilerParams(
            dimension_semantics=("arbitrary",),
            vmem_limit_bytes=int((64 << 20) * 0.9),
        ),
        cost_estimate=pl.CostEstimate(
            flops=2 * _N_HOPS * D * n_pad * n_pad,
            transcendentals=0,
            bytes_accessed=n_pad * n_pad * 2
            + n_pad * D * 4
            + (_N_HOPS + 1) * n_pad * D * 4,
        ),
    )(x0, m)
    if n_pad != N:
        out = out[:N]
    return out.astype(embed.dtype)


# speedup vs baseline: 1.6594x; 1.0101x over previous
import functools

import jax
import jax.numpy as jnp
from jax.experimental import pallas as pl
from jax.experimental.pallas import tpu as pltpu

_N_HOPS = 3


def _hops_kernel(x0_ref, m_ref, out_ref, state_ref):
    h = pl.program_id(0)

    @pl.when(h == 0)
    def _init():
        x0 = x0_ref[...]
        state_ref[...] = x0
        out_ref[:, 0, :] = x0

    @pl.when(h > 0)
    def _hop():
        new = jnp.dot(
            m_ref[...],
            state_ref[...].astype(jnp.bfloat16),
            preferred_element_type=jnp.float32,
        )
        out_ref[:, h, :] = new
        state_ref[...] = new


def _build_message_matrix(row, col, trend, n_pad):
    lin = col * n_pad + row
    E = lin.shape[0]
    if E % 16 == 0 and E >= 16:
        q = E // 16
        sizes = [q, q] + [2 * q] * 7
    else:
        sizes = [E]
    m = jnp.zeros((n_pad * n_pad,), dtype=jnp.float32)
    hist = []
    off = 0
    for i, c in enumerate(sizes):
        idx = lin[off : off + c]
        val = trend[off : off + c]
        if i >= 2:
            idx, _ = jax.lax.optimization_barrier((idx, hist[i - 2]))
        m = m.at[idx].add(val)
        hist.append(m)
        off += c
    return m.reshape(n_pad, n_pad)


def kernel(embed, adj_sp_norm, edge_index, edge_weight, trend):
    del adj_sp_norm, edge_weight
    N, D = embed.shape
    n_pad = (N + 127) // 128 * 128
    row, col = edge_index[0], edge_index[1]

    m = _build_message_matrix(row, col, trend, n_pad).astype(jnp.bfloat16)
    x0 = embed.astype(jnp.float32)
    if n_pad != N:
        x0 = jnp.pad(x0, ((0, n_pad - N), (0, 0)))

    out = pl.pallas_call(
        _hops_kernel,
        out_shape=jax.ShapeDtypeStruct((n_pad, _N_HOPS + 1, D), jnp.float32),
        grid_spec=pltpu.PrefetchScalarGridSpec(
            num_scalar_prefetch=0,
            grid=(_N_HOPS + 1,),
            in_specs=[
                pl.BlockSpec((n_pad, D), lambda h: (0, 0)),
                pl.BlockSpec((n_pad, n_pad), lambda h: (0, 0)),
            ],
            out_specs=pl.BlockSpec(
                (n_pad, _N_HOPS + 1, D), lambda h: (0, 0, 0)
            ),
            scratch_shapes=[pltpu.VMEM((n_pad, D), jnp.float32)],
        ),
        compiler_params=pltpu.CompilerParams(
            dimension_semantics=("arbitrary",),
            vmem_limit_bytes=int((64 << 20) * 0.9),
        ),
        cost_estimate=pl.CostEstimate(
            flops=2 * _N_HOPS * D * n_pad * n_pad,
            transcendentals=0,
            bytes_accessed=n_pad * n_pad * 2
            + n_pad * D * 4
            + (_N_HOPS + 1) * n_pad * D * 4,
        ),
    )(x0, m)
    if n_pad != N:
        out = out[:N]
    return out.astype(embed.dtype)


# speedup vs baseline: 1.6825x; 1.0139x over previous
import functools

import jax
import jax.numpy as jnp
from jax.experimental import pallas as pl
from jax.experimental.pallas import tpu as pltpu

_N_HOPS = 3


def _hops_kernel(x0_ref, m_ref, out_ref, state_ref):
    h = pl.program_id(0)

    @pl.when(h == 0)
    def _init():
        x0 = x0_ref[...]
        state_ref[...] = x0
        out_ref[:, 0, :] = x0

    @pl.when(h > 0)
    def _hop():
        new = jnp.dot(
            m_ref[...],
            state_ref[...].astype(jnp.bfloat16),
            preferred_element_type=jnp.float32,
        )
        out_ref[:, h, :] = new
        state_ref[...] = new


def _build_message_matrix(row, col, trend, n_pad):
    lin = col * n_pad + row
    E = lin.shape[0]
    if E % 32 == 0 and E >= 32:
        q = E // 32
        sizes = [q, 2 * q, 4 * q, 4 * q, 4 * q, 4 * q, 4 * q, 4 * q, 5 * q]
    else:
        sizes = [E]
    m = jnp.zeros((n_pad * n_pad,), dtype=jnp.float32)
    hist = []
    off = 0
    for i, c in enumerate(sizes):
        idx = lin[off : off + c]
        val = trend[off : off + c]
        if i >= 2:
            idx, _ = jax.lax.optimization_barrier((idx, hist[i - 2]))
        m = m.at[idx].add(val)
        hist.append(m)
        off += c
    return m.reshape(n_pad, n_pad)


def kernel(embed, adj_sp_norm, edge_index, edge_weight, trend):
    del adj_sp_norm, edge_weight
    N, D = embed.shape
    n_pad = (N + 127) // 128 * 128
    row, col = edge_index[0], edge_index[1]

    m = _build_message_matrix(row, col, trend, n_pad).astype(jnp.bfloat16)
    x0 = embed.astype(jnp.float32)
    if n_pad != N:
        x0 = jnp.pad(x0, ((0, n_pad - N), (0, 0)))

    out = pl.pallas_call(
        _hops_kernel,
        out_shape=jax.ShapeDtypeStruct((n_pad, _N_HOPS + 1, D), jnp.float32),
        grid_spec=pltpu.PrefetchScalarGridSpec(
            num_scalar_prefetch=0,
            grid=(_N_HOPS + 1,),
            in_specs=[
                pl.BlockSpec((n_pad, D), lambda h: (0, 0)),
                pl.BlockSpec((n_pad, n_pad), lambda h: (0, 0)),
            ],
            out_specs=pl.BlockSpec(
                (n_pad, _N_HOPS + 1, D), lambda h: (0, 0, 0)
            ),
            scratch_shapes=[pltpu.VMEM((n_pad, D), jnp.float32)],
        ),
        compiler_params=pltpu.CompilerParams(
            dimension_semantics=("arbitrary",),
            vmem_limit_bytes=int((64 << 20) * 0.9),
        ),
        cost_estimate=pl.CostEstimate(
            flops=2 * _N_HOPS * D * n_pad * n_pad,
            transcendentals=0,
            bytes_accessed=n_pad * n_pad * 2
            + n_pad * D * 4
            + (_N_HOPS + 1) * n_pad * D * 4,
        ),
    )(x0, m)
    if n_pad != N:
        out = out[:N]
    return out.astype(embed.dtype)


# speedup vs baseline: 1.6867x; 1.0025x over previous
import jax
import jax.numpy as jnp
from jax.experimental import pallas as pl
from jax.experimental.pallas import tpu as pltpu

_N_HOPS = 3


def _hops_kernel(x0_ref, m_ref, out_ref, state_ref):
    h = pl.program_id(0)

    @pl.when(h == 0)
    def _init():
        x0 = x0_ref[...]
        state_ref[...] = x0
        out_ref[:, 0, :] = x0

    @pl.when(h > 0)
    def _hop():
        new = jnp.dot(
            m_ref[...],
            state_ref[...].astype(jnp.bfloat16),
            preferred_element_type=jnp.float32,
        )
        out_ref[:, h, :] = new
        state_ref[...] = new


def _build_message_matrix(row, col, trend, n_pad):
    lin = col * n_pad + row
    E = lin.shape[0]
    if E % 32 == 0 and E >= 32:
        q = E // 32
        sizes = [q, 2 * q, 4 * q, 4 * q, 4 * q, 4 * q, 4 * q, 4 * q, 5 * q]
    else:
        sizes = [E]
    m = jnp.zeros((n_pad * n_pad,), dtype=jnp.float32)
    hist = []
    off = 0
    for i, c in enumerate(sizes):
        idx = lin[off : off + c]
        val = trend[off : off + c]
        if i >= 2:
            idx, _ = jax.lax.optimization_barrier((idx, hist[i - 2]))
        m = m.at[idx].add(val)
        hist.append(m)
        off += c
    return m.reshape(n_pad, n_pad)


def kernel(embed, adj_sp_norm, edge_index, edge_weight, trend):
    del adj_sp_norm, edge_weight
    N, D = embed.shape
    n_pad = (N + 127) // 128 * 128
    row, col = edge_index[0], edge_index[1]

    m = _build_message_matrix(row, col, trend, n_pad).astype(jnp.bfloat16)
    x0 = embed.astype(jnp.float32)
    if n_pad != N:
        x0 = jnp.pad(x0, ((0, n_pad - N), (0, 0)))

    out = pl.pallas_call(
        _hops_kernel,
        out_shape=jax.ShapeDtypeStruct((n_pad, _N_HOPS + 1, D), jnp.float32),
        grid_spec=pltpu.PrefetchScalarGridSpec(
            num_scalar_prefetch=0,
            grid=(_N_HOPS + 1,),
            in_specs=[
                pl.BlockSpec((n_pad, D), lambda h: (0, 0)),
                pl.BlockSpec((n_pad, n_pad), lambda h: (0, 0)),
            ],
            out_specs=pl.BlockSpec(
                (n_pad, _N_HOPS + 1, D), lambda h: (0, 0, 0)
            ),
            scratch_shapes=[pltpu.VMEM((n_pad, D), jnp.float32)],
        ),
        compiler_params=pltpu.CompilerParams(
            dimension_semantics=("arbitrary",),
            vmem_limit_bytes=int((64 << 20) * 0.9),
        ),
        cost_estimate=pl.CostEstimate(
            flops=2 * _N_HOPS * D * n_pad * n_pad,
            transcendentals=0,
            bytes_accessed=n_pad * n_pad * 2
            + n_pad * D * 4
            + (_N_HOPS + 1) * n_pad * D * 4,
        ),
    )(x0, m)
    if n_pad != N:
        out = out[:N]
    return out.astype(embed.dtype)
